# baseline (speedup 1.0000x reference)
import functools

import jax
import jax.numpy as jnp
from jax import lax
from jax.experimental import pallas as pl
from jax.experimental.pallas import tpu as pltpu
from jax.experimental.pallas import tpu_sc as plsc

NUM_SPEAKER = 100000
EMB_DIM = 64
BATCH = 4096

_info = plsc.get_sparse_core_info()
_NC, _NS = _info.num_cores, _info.num_subcores
_NW = _NC * _NS
_ROWS_PER_W = EMB_DIM // _NW


@functools.partial(
    pl.kernel,
    mesh=plsc.VectorSubcoreMesh(core_axis_name="c", subcore_axis_name="s"),
    out_type=jax.ShapeDtypeStruct((EMB_DIM, BATCH), jnp.float32),
    scratch_types=[
        pltpu.VMEM((NUM_SPEAKER,), jnp.float32),
        pltpu.VMEM((BATCH,), jnp.int32),
        pltpu.VMEM((BATCH,), jnp.float32),
        pltpu.SemaphoreType.DMA,
    ],
    compiler_params=pltpu.CompilerParams(
        use_tc_tiling_on_sc=True, needs_layout_passes=False
    ),
)
def _lane_gather_kernel(tab_t, sid_hbm, out_t, row_v, sid_v, out_v, sem):
    wid = lax.axis_index("s") * _NC + lax.axis_index("c")
    c0 = wid * _ROWS_PER_W

    cp = pltpu.async_copy(tab_t.at[c0], row_v, sem)
    pltpu.sync_copy(sid_hbm, sid_v)

    for r in range(_ROWS_PER_W):
        cp.wait()

        def body(i, carry):
            off = pl.multiple_of(i * 16, 16)
            idx = sid_v[pl.ds(off, 16)]
            out_v[pl.ds(off, 16)] = plsc.load_gather(row_v, [idx])
            return carry

        lax.fori_loop(0, BATCH // 16, body, 0, unroll=4)
        pltpu.sync_copy(out_v, out_t.at[c0 + r])
        if r + 1 < _ROWS_PER_W:
            cp = pltpu.async_copy(tab_t.at[c0 + r + 1], row_v, sem)


def kernel(sid, cropped_waveform, emb_table):
    del cropped_waveform
    out_t = _lane_gather_kernel(emb_table.T, sid.astype(jnp.int32))
    return out_t.T


# speedup vs baseline: 1.1217x; 1.1217x over previous
import functools

import jax
import jax.numpy as jnp
from jax import lax
from jax.experimental import pallas as pl
from jax.experimental.pallas import tpu as pltpu
from jax.experimental.pallas import tpu_sc as plsc

NUM_SPEAKER = 100000
EMB_DIM = 64
BATCH = 4096

_info = plsc.get_sparse_core_info()
_NC, _NS = _info.num_cores, _info.num_subcores
_NW = _NC * _NS
_ROWS_PER_W = EMB_DIM // _NW


@functools.partial(
    pl.kernel,
    mesh=plsc.VectorSubcoreMesh(core_axis_name="c", subcore_axis_name="s"),
    out_type=jax.ShapeDtypeStruct((EMB_DIM, BATCH), jnp.float32),
    scratch_types=[
        pltpu.VMEM((NUM_SPEAKER,), jnp.float32),
        pltpu.VMEM((BATCH,), jnp.int32),
        pltpu.VMEM((BATCH,), jnp.float32),
        pltpu.SemaphoreType.DMA,
    ],
    compiler_params=pltpu.CompilerParams(
        use_tc_tiling_on_sc=True, needs_layout_passes=False
    ),
)
def _lane_gather_kernel(tab_t, sid_hbm, out_t, row_v, sid_v, out_v, sem):
    wid = lax.axis_index("s") * _NC + lax.axis_index("c")
    c0 = wid * _ROWS_PER_W

    cp = pltpu.async_copy(tab_t.at[c0], row_v, sem)
    pltpu.sync_copy(sid_hbm, sid_v)

    for r in range(_ROWS_PER_W):
        cp.wait()

        @plsc.parallel_loop(0, BATCH, step=16, unroll=4)
        def _(i):
            off = pl.multiple_of(i, 16)
            idx = sid_v[pl.ds(off, 16)]
            out_v[pl.ds(off, 16)] = plsc.load_gather(row_v, [idx])
        pltpu.sync_copy(out_v, out_t.at[c0 + r])
        if r + 1 < _ROWS_PER_W:
            cp = pltpu.async_copy(tab_t.at[c0 + r + 1], row_v, sem)


def kernel(sid, cropped_waveform, emb_table):
    del cropped_waveform
    out_t = _lane_gather_kernel(emb_table.T, sid.astype(jnp.int32))
    return out_t.T


# speedup vs baseline: 1.1221x; 1.0003x over previous
import functools

import jax
import jax.numpy as jnp
from jax import lax
from jax.experimental import pallas as pl
from jax.experimental.pallas import tpu as pltpu
from jax.experimental.pallas import tpu_sc as plsc

NUM_SPEAKER = 100000
EMB_DIM = 64
BATCH = 4096

_info = plsc.get_sparse_core_info()
_NC, _NS = _info.num_cores, _info.num_subcores
_NW = _NC * _NS
_ROWS_PER_W = EMB_DIM // _NW


@functools.partial(
    pl.kernel,
    mesh=plsc.VectorSubcoreMesh(core_axis_name="c", subcore_axis_name="s"),
    out_type=jax.ShapeDtypeStruct((EMB_DIM, BATCH), jnp.float32),
    scratch_types=[
        pltpu.VMEM((NUM_SPEAKER,), jnp.float32),
        pltpu.VMEM((BATCH,), jnp.int32),
        pltpu.VMEM((BATCH,), jnp.float32),
        pltpu.SemaphoreType.DMA,
    ],
    compiler_params=pltpu.CompilerParams(
        use_tc_tiling_on_sc=True, needs_layout_passes=False
    ),
)
def _lane_gather_kernel(tab_t, sid_hbm, out_t, row_v, sid_v, out_v, sem):
    wid = lax.axis_index("s") * _NC + lax.axis_index("c")
    c0 = wid * _ROWS_PER_W

    cp = pltpu.async_copy(tab_t.at[c0], row_v, sem)
    pltpu.sync_copy(sid_hbm, sid_v)

    for r in range(_ROWS_PER_W):
        cp.wait()

        @plsc.parallel_loop(0, BATCH, step=16, unroll=8)
        def _(i):
            off = pl.multiple_of(i, 16)
            idx = sid_v[pl.ds(off, 16)]
            out_v[pl.ds(off, 16)] = plsc.load_gather(row_v, [idx])
        pltpu.sync_copy(out_v, out_t.at[c0 + r])
        if r + 1 < _ROWS_PER_W:
            cp = pltpu.async_copy(tab_t.at[c0 + r + 1], row_v, sem)


def kernel(sid, cropped_waveform, emb_table):
    del cropped_waveform
    out_t = _lane_gather_kernel(emb_table.T, sid.astype(jnp.int32))
    return out_t.T
